# baseline (speedup 1.0000x reference)
import dataclasses
import functools

import jax
import jax.numpy as jnp
from jax import lax
from jax.experimental import pallas as pl
from jax.experimental.pallas import tpu as pltpu
from jax.experimental.pallas import tpu_sc as plsc

NC = 2
NS = 16
L = 16


def _matmul_table(feat, weight):
    N, IN = feat.shape
    R, _, OUT = weight.shape

    def body(feat_ref, w_ref, y_ref):
        y_ref[...] = jnp.dot(feat_ref[...], w_ref[...],
                             preferred_element_type=jnp.float32)

    return pl.pallas_call(
        body,
        grid=(R,),
        in_specs=[
            pl.BlockSpec((N, IN), lambda r: (0, 0)),
            pl.BlockSpec((None, IN, OUT), lambda r: (r, 0, 0)),
        ],
        out_specs=pl.BlockSpec((None, N, OUT), lambda r: (r, 0, 0)),
        out_shape=jax.ShapeDtypeStruct((R, N, OUT), jnp.float32),
    )(feat, weight)


def _sum_partials(parts, n_out):
    _, NP, OUT = parts.shape

    def body(p_ref, o_ref):
        o_ref[...] = p_ref[0, :n_out, :] + p_ref[1, :n_out, :]

    return pl.pallas_call(
        body,
        out_shape=jax.ShapeDtypeStruct((n_out, OUT), jnp.float32),
    )(parts)


def _sc_edge_kernel(y_flat, meta, normc, zeros_nd, n_nodes):
    RN, OUT = y_flat.shape
    total_chunks, _, k_chunk = meta.shape
    n_tiles = NC * NS
    n_chunks = total_chunks // n_tiles
    ep_tile = n_chunks * k_chunk
    rows_per_s = n_nodes // NS

    mesh = plsc.VectorSubcoreMesh(core_axis_name="c", subcore_axis_name="s")

    cp = pltpu.CompilerParams()
    if "needs_layout_passes" in pltpu.CompilerParams.__dataclass_fields__:
        cp = dataclasses.replace(cp, needs_layout_passes=False)

    nbuf = 4
    assert n_chunks % nbuf == 0

    @functools.partial(
        pl.kernel,
        compiler_params=cp,
        out_type=jax.ShapeDtypeStruct((NC, n_nodes, OUT), jnp.float32),
        mesh=mesh,
        scratch_types=(
            [pltpu.VMEM((2, k_chunk), jnp.int32) for _ in range(nbuf)]
            + [pltpu.VMEM((k_chunk,), jnp.float32) for _ in range(nbuf)]
            + [pltpu.VMEM((k_chunk, OUT), jnp.float32) for _ in range(nbuf)]
            + [pltpu.VMEM_SHARED((n_nodes, OUT), jnp.float32)]
            + [pltpu.SemaphoreType.DMA for _ in range(nbuf)]
        ),
    )
    def k(y_hbm, meta_hbm, normc_hbm, z_hbm, part_hbm, *scr):
        metas = scr[0:nbuf]
        norms = scr[nbuf:2 * nbuf]
        rows = scr[2 * nbuf:3 * nbuf]
        acc_sh = scr[3 * nbuf]
        sems = scr[3 * nbuf + 1:3 * nbuf + 1 + nbuf]

        c = lax.axis_index("c")
        s = lax.axis_index("s")
        wid = c * NS + s
        cbase = wid * n_chunks

        pltpu.sync_copy(z_hbm.at[pl.ds(s * rows_per_s, rows_per_s)],
                        acc_sh.at[pl.ds(s * rows_per_s, rows_per_s)])
        plsc.subcore_barrier()

        def meta_sync(ci, meta_v, norm_v):
            pltpu.sync_copy(meta_hbm.at[cbase + ci], meta_v)
            pltpu.sync_copy(normc_hbm.at[cbase + ci], norm_v)

        def gather_start(meta_v, rows_v, sem):
            pltpu.async_copy(y_hbm.at[meta_v.at[0]], rows_v, sem)

        def gather_wait(meta_v, rows_v, sem):
            pltpu.make_async_copy(y_hbm.at[meta_v.at[0]], rows_v, sem).wait()

        def scale_and_scatter(meta_v, norm_v, rows_v):
            @plsc.parallel_loop(0, k_chunk, unroll=4)
            def _(e):
                nv = plsc.load_gather(norm_v, [jnp.full((L,), e, jnp.int32)])
                for j in range(OUT // L):
                    sl = (e, pl.ds(j * L, L))
                    rows_v[sl] = rows_v[sl] * nv

            pltpu.sync_copy(rows_v, acc_sh.at[meta_v.at[1]], add=True)

        for b in range(nbuf):
            meta_sync(b, metas[b], norms[b])
            gather_start(metas[b], rows[b], sems[b])

        @pl.loop(0, n_chunks // nbuf)
        def _(h):
            for b in range(nbuf):
                ci = h * nbuf + b
                gather_wait(metas[b], rows[b], sems[b])
                scale_and_scatter(metas[b], norms[b], rows[b])

                @pl.when(ci + nbuf < n_chunks)
                def _():
                    meta_sync(ci + nbuf, metas[b], norms[b])
                    gather_start(metas[b], rows[b], sems[b])

        plsc.subcore_barrier()
        pltpu.sync_copy(acc_sh.at[pl.ds(s * rows_per_s, rows_per_s)],
                        part_hbm.at[c, pl.ds(s * rows_per_s, rows_per_s)])

    return k(y_flat, meta, normc, zeros_nd)


def kernel(feat, edge_index, etypes, norm, weight):
    N, IN = feat.shape
    R, _, OUT = weight.shape
    E = edge_index.shape[1]

    src = edge_index[0]
    dst = edge_index[1]

    y = _matmul_table(feat, weight)
    y_flat = y.reshape(R * N, OUT)

    k_chunk = 64
    n_tiles = NC * NS
    quantum = n_tiles * k_chunk * 4
    ep = ((E + quantum - 1) // quantum) * quantum
    pad = ep - E
    gidx = etypes.astype(jnp.int32) * N + src.astype(jnp.int32)
    gidx = jnp.concatenate([gidx, jnp.zeros((pad,), jnp.int32)])
    dstp = jnp.concatenate([dst.astype(jnp.int32), jnp.zeros((pad,), jnp.int32)])
    normf = jnp.concatenate([norm.reshape(E).astype(jnp.float32),
                             jnp.zeros((pad,), jnp.float32)])
    meta = jnp.stack([gidx.reshape(-1, k_chunk), dstp.reshape(-1, k_chunk)],
                     axis=1)
    normc = normf.reshape(-1, k_chunk)

    n_pad = ((N + NS * 8 - 1) // (NS * 8)) * (NS * 8)
    zeros_nd = jnp.zeros((n_pad, OUT), jnp.float32)

    parts = _sc_edge_kernel(y_flat, meta, normc, zeros_nd, n_pad)

    return _sum_partials(parts, N)


# speedup vs baseline: 1.7685x; 1.7685x over previous
import dataclasses
import functools

import numpy as np

import jax
import jax.numpy as jnp
from jax import lax
from jax.experimental import pallas as pl
from jax.experimental.pallas import tpu as pltpu
from jax.experimental.pallas import tpu_sc as plsc

NC = 2
NS = 16
L = 16


def _matmul_table(feat, weight_a, weight_b):
    N, IN = feat.shape
    R, _, H = weight_a.shape

    def body(feat_ref, wa_ref, wb_ref, y_ref):
        a = jnp.dot(feat_ref[...], wa_ref[...],
                    preferred_element_type=jnp.float32)
        b = jnp.dot(feat_ref[...], wb_ref[...],
                    preferred_element_type=jnp.float32)
        abits = jax.lax.bitcast_convert_type(
            a.astype(jnp.bfloat16).astype(jnp.float32), jnp.int32)
        bbits = jax.lax.bitcast_convert_type(
            b.astype(jnp.bfloat16).astype(jnp.float32), jnp.int32)
        y_ref[...] = bbits | jax.lax.shift_right_logical(abits, 16)

    return pl.pallas_call(
        body,
        grid=(R,),
        in_specs=[
            pl.BlockSpec((N, IN), lambda r: (0, 0)),
            pl.BlockSpec((None, IN, H), lambda r: (r, 0, 0)),
            pl.BlockSpec((None, IN, H), lambda r: (r, 0, 0)),
        ],
        out_specs=pl.BlockSpec((None, N, H), lambda r: (r, 0, 0)),
        out_shape=jax.ShapeDtypeStruct((R, N, H), jnp.int32),
    )(feat, weight_a, weight_b)


def _sum_partials(parts, n_out):
    _, NP, OUT = parts.shape

    def body(p_ref, o_ref):
        o_ref[...] = p_ref[0, :n_out, :] + p_ref[1, :n_out, :]

    return pl.pallas_call(
        body,
        out_shape=jax.ShapeDtypeStruct((n_out, OUT), jnp.float32),
    )(parts)


def _sc_edge_kernel(y_flat, meta, normc, zeros_nd, n_nodes):
    RN, H = y_flat.shape
    OUT = 2 * H
    total_chunks, _, k_chunk = meta.shape
    n_tiles = NC * NS
    n_chunks = total_chunks // n_tiles
    ep_tile = n_chunks * k_chunk
    rows_per_s = n_nodes // NS

    mesh = plsc.VectorSubcoreMesh(core_axis_name="c", subcore_axis_name="s")

    cp = pltpu.CompilerParams()
    if "needs_layout_passes" in pltpu.CompilerParams.__dataclass_fields__:
        cp = dataclasses.replace(cp, needs_layout_passes=False)
    if "use_tc_tiling_on_sc" in pltpu.CompilerParams.__dataclass_fields__:
        cp = dataclasses.replace(cp, use_tc_tiling_on_sc=False)

    nbuf = 2
    assert n_chunks % nbuf == 0

    @functools.partial(
        pl.kernel,
        compiler_params=cp,
        out_type=jax.ShapeDtypeStruct((NC, n_nodes, OUT), jnp.float32),
        mesh=mesh,
        scratch_types=(
            [pltpu.VMEM((2, k_chunk), jnp.int32) for _ in range(nbuf)]
            + [pltpu.VMEM((k_chunk,), jnp.float32) for _ in range(nbuf)]
            + [pltpu.VMEM((k_chunk, H), jnp.int32) for _ in range(nbuf)]
            + [pltpu.VMEM((k_chunk, OUT), jnp.float32)]
            + [pltpu.VMEM_SHARED((n_nodes, OUT), jnp.float32)]
            + [pltpu.SemaphoreType.DMA for _ in range(nbuf)]
        ),
    )
    def k(y_hbm, meta_hbm, normc_hbm, z_hbm, part_hbm, *scr):
        metas = scr[0:nbuf]
        norms = scr[nbuf:2 * nbuf]
        rows = scr[2 * nbuf:3 * nbuf]
        stage_v = scr[3 * nbuf]
        acc_sh = scr[3 * nbuf + 1]
        sems = scr[3 * nbuf + 2:3 * nbuf + 2 + nbuf]

        c = lax.axis_index("c")
        s = lax.axis_index("s")
        wid = c * NS + s
        cbase = wid * n_chunks

        pltpu.sync_copy(z_hbm.at[pl.ds(s * rows_per_s, rows_per_s)],
                        acc_sh.at[pl.ds(s * rows_per_s, rows_per_s)])
        plsc.subcore_barrier()

        def meta_sync(ci, meta_v, norm_v):
            pltpu.sync_copy(meta_hbm.at[cbase + ci], meta_v)
            pltpu.sync_copy(normc_hbm.at[cbase + ci], norm_v)

        def gather_start(meta_v, rows_v, sem):
            pltpu.async_copy(y_hbm.at[meta_v.at[0]], rows_v, sem)

        def gather_wait(meta_v, rows_v, sem):
            pltpu.make_async_copy(y_hbm.at[meta_v.at[0]], rows_v, sem).wait()

        def scale_and_scatter(meta_v, norm_v, rows_v):
            @plsc.parallel_loop(0, k_chunk, unroll=4)
            def _(e):
                nv = plsc.load_gather(norm_v, [jnp.full((L,), e, jnp.int32)])
                for j in range(OUT // (2 * L)):
                    w16 = rows_v[e, pl.ds(j * L, L)]
                    v = plsc.bitcast(w16, jnp.bfloat16)
                    a, b = plsc.unpack(v, format=plsc.PackFormat.INTERLEAVED)
                    stage_v[e, pl.ds(j * 2 * L, L)] = a * nv
                    stage_v[e, pl.ds(j * 2 * L + L, L)] = b * nv

            pltpu.sync_copy(stage_v, acc_sh.at[meta_v.at[1]], add=True)

        for b in range(nbuf):
            meta_sync(b, metas[b], norms[b])
            gather_start(metas[b], rows[b], sems[b])

        @pl.loop(0, n_chunks // nbuf)
        def _(h):
            for b in range(nbuf):
                ci = h * nbuf + b
                gather_wait(metas[b], rows[b], sems[b])
                scale_and_scatter(metas[b], norms[b], rows[b])

                @pl.when(ci + nbuf < n_chunks)
                def _():
                    meta_sync(ci + nbuf, metas[b], norms[b])
                    gather_start(metas[b], rows[b], sems[b])

        plsc.subcore_barrier()
        pltpu.sync_copy(acc_sh.at[pl.ds(s * rows_per_s, rows_per_s)],
                        part_hbm.at[c, pl.ds(s * rows_per_s, rows_per_s)])

    return k(y_flat, meta, normc, zeros_nd)


def kernel(feat, edge_index, etypes, norm, weight):
    N, IN = feat.shape
    R, _, OUT = weight.shape
    E = edge_index.shape[1]

    src = edge_index[0]
    dst = edge_index[1]

    i16 = np.arange(16)
    qa = np.concatenate([32 * j + i16 for j in range(OUT // 32)])
    qb = qa + 16

    y = _matmul_table(feat, weight[:, :, qa], weight[:, :, qb])
    y_flat = y.reshape(R * N, OUT // 2)

    k_chunk = 128
    n_tiles = NC * NS
    quantum = n_tiles * k_chunk * 2
    ep = ((E + quantum - 1) // quantum) * quantum
    pad = ep - E
    gidx = etypes.astype(jnp.int32) * N + src.astype(jnp.int32)
    gidx = jnp.concatenate([gidx, jnp.zeros((pad,), jnp.int32)])
    dstp = jnp.concatenate([dst.astype(jnp.int32), jnp.zeros((pad,), jnp.int32)])
    normf = jnp.concatenate([norm.reshape(E).astype(jnp.float32),
                             jnp.zeros((pad,), jnp.float32)])
    meta = jnp.stack([gidx.reshape(-1, k_chunk), dstp.reshape(-1, k_chunk)],
                     axis=1)
    normc = normf.reshape(-1, k_chunk)

    n_pad = ((N + NS * 8 - 1) // (NS * 8)) * (NS * 8)
    zeros_nd = jnp.zeros((n_pad, OUT), jnp.float32)

    parts = _sc_edge_kernel(y_flat, meta, normc, zeros_nd, n_pad)

    return _sum_partials(parts, N)
